# speedup vs baseline: 1427.6520x; 1427.6520x over previous
import numpy as np
import jax
import jax.numpy as jnp
from jax.experimental import pallas as pl
from jax.experimental.pallas import tpu as pltpu

_M = 100
_T = [float(v) for v in np.linspace(0.0, 1.0, _M).astype(np.float32)]


def _roc_kernel(x_ref, y_ref, roc_ref, dcv_ref, a1_ref, a2_ref, acc_ref):
    n = pl.program_id(1)
    n_last = pl.num_programs(1) - 1
    xb = x_ref[...]
    br = xb.shape[0]

    lane = jax.lax.broadcasted_iota(jnp.int32, (br, 128), 1)
    upd = jnp.zeros((br, 128), jnp.float32)
    for m in range(_M):
        masked = jnp.where(xb > _T[m], xb, 0.0)
        sm = jnp.sum(masked, axis=1, keepdims=True)
        upd = jnp.where(lane == m, sm, upd)

    @pl.when(n == 0)
    def _():
        acc_ref[...] = upd

    @pl.when(n > 0)
    def _():
        acc_ref[...] = acc_ref[...] + upd

    @pl.when(n == n_last)
    def _():
        s = acc_ref[...]
        inv_y = 1.0 / y_ref[...]
        roc = s[:, 0:_M] * inv_y
        binsm = (s[:, 0 : _M - 1] - s[:, 1:_M]) * inv_y
        dlast = (binsm[:, _M - 2 : _M - 1] + binsm[:, _M - 3 : _M - 2]) * 0.5
        deriv = jnp.concatenate([binsm, dlast], axis=1)
        roc_ref[...] = roc
        dcv_ref[...] = deriv
        a1_ref[...] = jnp.sum(roc, axis=1, keepdims=True) - 0.5 * (
            roc[:, 0:1] + roc[:, _M - 1 : _M]
        )
        a2_ref[...] = jnp.sum(deriv, axis=1, keepdims=True) - 0.5 * (
            deriv[:, 0:1] + deriv[:, _M - 1 : _M]
        )


def kernel(x, y):
    b, n = x.shape
    br, nn = 8, 2048
    grid = (b // br, n // nn)
    roc, deriv, a1, a2 = pl.pallas_call(
        _roc_kernel,
        grid=grid,
        in_specs=[
            pl.BlockSpec((br, nn), lambda i, j: (i, j)),
            pl.BlockSpec((br, 1), lambda i, j: (i, 0)),
        ],
        out_specs=[
            pl.BlockSpec((br, _M), lambda i, j: (i, 0)),
            pl.BlockSpec((br, _M), lambda i, j: (i, 0)),
            pl.BlockSpec((br, 1), lambda i, j: (i, 0)),
            pl.BlockSpec((br, 1), lambda i, j: (i, 0)),
        ],
        out_shape=[
            jax.ShapeDtypeStruct((b, _M), jnp.float32),
            jax.ShapeDtypeStruct((b, _M), jnp.float32),
            jax.ShapeDtypeStruct((b, 1), jnp.float32),
            jax.ShapeDtypeStruct((b, 1), jnp.float32),
        ],
        scratch_shapes=[pltpu.VMEM((br, 128), jnp.float32)],
        compiler_params=pltpu.CompilerParams(
            dimension_semantics=("parallel", "arbitrary")
        ),
    )(x, y.reshape(b, 1))
    return (roc, deriv, a1.reshape(b), a2.reshape(b))
